# baseline (speedup 1.0000x reference)
import functools

import jax
import jax.numpy as jnp
from jax import lax
from jax.experimental import pallas as pl
from jax.experimental.pallas import tpu as pltpu
from jax.experimental.pallas import tpu_sc as plsc

_B = 4096
_D = 16


def _gather_sc(P, Q, user_id, item_id):
    info = plsc.get_sparse_core_info()
    nc, ns = info.num_cores, info.num_subcores
    nw = nc * ns
    b_per_w = _B // nw

    mesh = plsc.VectorSubcoreMesh(core_axis_name="c", subcore_axis_name="s")

    @functools.partial(
        pl.kernel,
        mesh=mesh,
        out_type=[
            jax.ShapeDtypeStruct((_B, _D), jnp.float32),
            jax.ShapeDtypeStruct((_B, _D), jnp.float32),
        ],
        scratch_types=[
            pltpu.VMEM((b_per_w,), jnp.int32),
            pltpu.VMEM((b_per_w,), jnp.int32),
            pltpu.VMEM((b_per_w, _D), jnp.float32),
            pltpu.VMEM((b_per_w, _D), jnp.float32),
            pltpu.SemaphoreType.DMA,
            pltpu.SemaphoreType.DMA,
        ],
        compiler_params=pltpu.CompilerParams(use_tc_tiling_on_sc=False),
    )
    def gather(p_hbm, q_hbm, uid_hbm, iid_hbm, pu_hbm, qi_hbm,
               uidx_v, iidx_v, prow_v, qrow_v, psem, qsem):
        wid = lax.axis_index("s") * nc + lax.axis_index("c")
        base = wid * b_per_w
        pltpu.sync_copy(uid_hbm.at[pl.ds(base, b_per_w)], uidx_v)
        pltpu.sync_copy(iid_hbm.at[pl.ds(base, b_per_w)], iidx_v)
        pcopy = pltpu.async_copy(p_hbm.at[uidx_v], prow_v, psem)
        qcopy = pltpu.async_copy(q_hbm.at[iidx_v], qrow_v, qsem)
        pcopy.wait()
        pltpu.sync_copy(prow_v, pu_hbm.at[pl.ds(base, b_per_w)])
        qcopy.wait()
        pltpu.sync_copy(qrow_v, qi_hbm.at[pl.ds(base, b_per_w)])

    return gather(P, Q, user_id, item_id)


def _matmul_tc(P_u, Q_i, tm=512):
    def body(p_ref, q_ref, o_ref):
        o_ref[...] = lax.dot_general(
            p_ref[...], q_ref[...],
            dimension_numbers=(((1,), (1,)), ((), ())),
            preferred_element_type=jnp.float32,
        )

    return pl.pallas_call(
        body,
        grid=(_B // tm,),
        in_specs=[
            pl.BlockSpec((tm, _D), lambda i: (i, 0)),
            pl.BlockSpec((_B, _D), lambda i: (0, 0)),
        ],
        out_specs=pl.BlockSpec((tm, _B), lambda i: (i, 0)),
        out_shape=jax.ShapeDtypeStruct((_B, _B), jnp.float32),
    )(P_u, Q_i)


def kernel(user_id, item_id, P, Q):
    P_u, Q_i = _gather_sc(P, Q, user_id, item_id)
    return _matmul_tc(P_u, Q_i)
